# Initial kernel scaffold; baseline (speedup 1.0000x reference)
import jax
import jax.numpy as jnp
from jax.experimental import pallas as pl


def kernel(atomic_numbers, d_ij, pair_indices, frequencies, emb_table, W_rbf, b_rbf, W_dense, b_dense):
    raise NotImplementedError("write your pallas kernel here")


# speedup vs baseline: 4.4648x; 4.4648x over previous
import jax
import jax.numpy as jnp
from jax import lax
from jax.experimental import pallas as pl
from jax.experimental.pallas import tpu as pltpu
from jax.experimental.pallas import tpu_sc as plsc

_RADIAL_CUTOFF = 5.0
_P = 6.0
_EA = -((_P + 1.0) * (_P + 2.0)) / 2.0
_EB = _P * (_P + 2.0)
_EC = -_P * (_P + 1.0) / 2.0

_CHUNK = 2000
_BE = 512


def _sc_zgather(pairs, atoms):
    n_edges = pairs.shape[0] // 2
    n_atoms = atoms.shape[0]
    info = plsc.get_sparse_core_info()
    nw = info.num_cores * info.num_subcores
    per_worker = n_edges // nw
    n_chunks = per_worker // _CHUNK

    def body(pairs_hbm, atoms_hbm, zi_hbm, zj_hbm, atoms_v, idx_v, z_v):
        wid = lax.axis_index("s") * info.num_cores + lax.axis_index("c")
        pltpu.sync_copy(atoms_hbm, atoms_v)
        for r, out_hbm in ((0, zi_hbm), (1, zj_hbm)):
            def chunk_body(c, _):
                base = wid * per_worker + c * _CHUNK
                pltpu.sync_copy(pairs_hbm.at[pl.ds(r * n_edges + base, _CHUNK)],
                                idx_v)
                for k in range(_CHUNK // 16):
                    iv = idx_v[pl.ds(k * 16, 16)]
                    z_v[pl.ds(k * 16, 16)] = plsc.load_gather(atoms_v, [iv])
                pltpu.sync_copy(z_v, out_hbm.at[pl.ds(base, _CHUNK)])
                return 0

            lax.fori_loop(0, n_chunks, chunk_body, 0)

    mesh = plsc.VectorSubcoreMesh(core_axis_name="c", subcore_axis_name="s")
    fn = pl.kernel(
        body,
        mesh=mesh,
        compiler_params=pltpu.CompilerParams(needs_layout_passes=False),
        out_type=[
            jax.ShapeDtypeStruct((n_edges,), jnp.int32),
            jax.ShapeDtypeStruct((n_edges,), jnp.int32),
        ],
        scratch_types=[
            pltpu.VMEM((n_atoms,), jnp.int32),
            pltpu.VMEM((_CHUNK,), jnp.int32),
            pltpu.VMEM((_CHUNK,), jnp.int32),
        ],
    )
    return fn(pairs, atoms)


def _tc_body(d_ref, zi_ref, zj_ref, freq_ref, emb_ref, wr_ref, br_ref,
             wd_ref, bd_ref, out_ref):
    x = d_ref[:, :] * (1.0 / _RADIAL_CUTOFF)
    x2 = x * x
    x5 = x2 * x2 * x
    env = 1.0 / x + x5 * (_EA + x * (_EB + x * _EC))
    env = jnp.where(x < 1.0, env, jnp.zeros_like(env))
    rbf = env * jnp.sin(x * freq_ref[:, :])
    h = jnp.dot(rbf, wr_ref[:, :], preferred_element_type=jnp.float32)
    h = h + br_ref[:, :]
    h = h * jax.nn.sigmoid(h)
    iota = lax.broadcasted_iota(jnp.int32, (_BE, 128), 1)
    oi = (zi_ref[:, :] == iota).astype(jnp.float32)
    oj = (zj_ref[:, :] == iota).astype(jnp.float32)
    xi = jnp.dot(oi, emb_ref[:, :], preferred_element_type=jnp.float32)
    xj = jnp.dot(oj, emb_ref[:, :], preferred_element_type=jnp.float32)
    acc = (jnp.dot(xi, wd_ref[0:32, :], preferred_element_type=jnp.float32)
           + jnp.dot(xj, wd_ref[32:64, :], preferred_element_type=jnp.float32)
           + jnp.dot(h, wd_ref[64:96, :], preferred_element_type=jnp.float32)
           + bd_ref[:, :])
    out_ref[:, :] = acc * jax.nn.sigmoid(acc)


def kernel(atomic_numbers, d_ij, pair_indices, frequencies, emb_table,
           W_rbf, b_rbf, W_dense, b_dense):
    n_edges = d_ij.shape[0]
    emb_size = emb_table.shape[1]
    zi, zj = _sc_zgather(pair_indices.astype(jnp.int32).reshape(-1),
                         atomic_numbers.astype(jnp.int32))
    embp = jnp.pad(emb_table, ((0, 128 - emb_table.shape[0]), (0, 0)))
    out = pl.pallas_call(
        _tc_body,
        grid=(n_edges // _BE,),
        in_specs=[
            pl.BlockSpec((_BE, 1), lambda i: (i, 0)),
            pl.BlockSpec((_BE, 1), lambda i: (i, 0)),
            pl.BlockSpec((_BE, 1), lambda i: (i, 0)),
            pl.BlockSpec((1, 128), lambda i: (0, 0)),
            pl.BlockSpec((128, emb_size), lambda i: (0, 0)),
            pl.BlockSpec((128, emb_size), lambda i: (0, 0)),
            pl.BlockSpec((1, emb_size), lambda i: (0, 0)),
            pl.BlockSpec((96, emb_size), lambda i: (0, 0)),
            pl.BlockSpec((1, emb_size), lambda i: (0, 0)),
        ],
        out_specs=pl.BlockSpec((_BE, emb_size), lambda i: (i, 0)),
        out_shape=jax.ShapeDtypeStruct((n_edges, emb_size), jnp.float32),
    )(d_ij, zi.reshape(n_edges, 1), zj.reshape(n_edges, 1),
      jnp.pad(frequencies, (0, 112)).reshape(1, 128), embp,
      jnp.pad(W_rbf, ((0, 112), (0, 0))), b_rbf.reshape(1, emb_size),
      W_dense, b_dense.reshape(1, emb_size))
    return out


# speedup vs baseline: 15.0667x; 3.3745x over previous
import jax
import jax.numpy as jnp
from jax import lax
from jax.experimental import pallas as pl
from jax.experimental.pallas import tpu as pltpu
from jax.experimental.pallas import tpu_sc as plsc

_RADIAL_CUTOFF = 5.0
_P = 6.0
_EA = -((_P + 1.0) * (_P + 2.0)) / 2.0
_EB = _P * (_P + 2.0)
_EC = -_P * (_P + 1.0) / 2.0

_CHUNK = 2000
_BE = 12800


def _sc_zgather(pairs, atoms):
    n_edges = pairs.shape[0] // 2
    n_atoms = atoms.shape[0]
    info = plsc.get_sparse_core_info()
    nw = info.num_cores * info.num_subcores
    per_worker = n_edges // nw
    n_chunks = per_worker // _CHUNK

    def body(pairs_hbm, atoms_hbm, zi_hbm, zj_hbm, atoms_v, idx_v, z_v):
        wid = lax.axis_index("s") * info.num_cores + lax.axis_index("c")
        pltpu.sync_copy(atoms_hbm, atoms_v)
        for r, out_hbm in ((0, zi_hbm), (1, zj_hbm)):
            def chunk_body(c, _):
                base = wid * per_worker + c * _CHUNK
                pltpu.sync_copy(pairs_hbm.at[pl.ds(r * n_edges + base, _CHUNK)],
                                idx_v)
                for k in range(_CHUNK // 16):
                    iv = idx_v[pl.ds(k * 16, 16)]
                    z_v[pl.ds(k * 16, 16)] = plsc.load_gather(atoms_v, [iv])
                pltpu.sync_copy(z_v, out_hbm.at[pl.ds(base, _CHUNK)])
                return 0

            lax.fori_loop(0, n_chunks, chunk_body, 0)

    mesh = plsc.VectorSubcoreMesh(core_axis_name="c", subcore_axis_name="s")
    fn = pl.kernel(
        body,
        mesh=mesh,
        compiler_params=pltpu.CompilerParams(needs_layout_passes=False),
        out_type=[
            jax.ShapeDtypeStruct((n_edges,), jnp.int32),
            jax.ShapeDtypeStruct((n_edges,), jnp.int32),
        ],
        scratch_types=[
            pltpu.VMEM((n_atoms,), jnp.int32),
            pltpu.VMEM((_CHUNK,), jnp.int32),
            pltpu.VMEM((_CHUNK,), jnp.int32),
        ],
    )
    return fn(pairs, atoms)


_S1 = 9.9993896014e-01
_S3 = -1.6614390484e-01
_S5 = 7.6898124879e-03
_PI = 3.14159265358979
_INV_PI = 1.0 / _PI
_CT = (((0,), (0,)), ((), ()))


def _dot(a, b):
    return lax.dot_general(a, b, _CT, preferred_element_type=jnp.float32)


def _silu(v):
    return (0.5 * v) * (1.0 + jnp.tanh(0.5 * v))


def _tc_body(d_ref, zi_ref, zj_ref, freq_ref, emb_ref, wr_ref, br_ref,
             wd_ref, bd_ref, out_ref):
    f32 = jnp.float32
    beq = d_ref.shape[2]
    emb = emb_ref[:, :]
    t1 = jnp.dot(emb, wd_ref[0:32, :],
                 preferred_element_type=f32).astype(jnp.bfloat16)
    t2 = jnp.dot(emb, wd_ref[32:64, :],
                 preferred_element_type=f32).astype(jnp.bfloat16)
    iota = lax.broadcasted_iota(jnp.int32, (96, beq), 0)
    parts = []
    for a in range(4):
        d = d_ref[0, a:a + 1, :]
        x = d * (1.0 / _RADIAL_CUTOFF)
        x2 = x * x
        x5 = x2 * x2 * x
        env = 1.0 / x + x5 * (_EA + x * (_EB + x * _EC))
        env = jnp.where(x < 1.0, env, jnp.zeros_like(env))
        tT = jnp.broadcast_to(x, (16, beq)) * freq_ref[:, :]
        n = (tT * _INV_PI + 0.5).astype(jnp.int32)
        r = tT - n.astype(f32) * _PI
        u = r * r
        s = r * (_S1 + u * (_S3 + u * _S5))
        s = jnp.where((n & 1) == 0, s, -s)
        rbfT = (jnp.broadcast_to(env, s.shape) * s).astype(jnp.bfloat16)
        hT = _silu(_dot(wr_ref[:, :], rbfT) + br_ref[:, :])
        oiT = (zi_ref[0, a:a + 1, :] == iota).astype(jnp.bfloat16)
        ojT = (zj_ref[0, a:a + 1, :] == iota).astype(jnp.bfloat16)
        accT = (_dot(t1, oiT) + _dot(t2, ojT)
                + _dot(wd_ref[64:96, :], hT.astype(jnp.bfloat16))
                + bd_ref[:, :])
        parts.append(_silu(accT))
    packedT = jnp.concatenate(parts, axis=0)
    out_ref[:, :] = packedT.T


def kernel(atomic_numbers, d_ij, pair_indices, frequencies, emb_table,
           W_rbf, b_rbf, W_dense, b_dense):
    n_edges = d_ij.shape[0]
    emb_size = emb_table.shape[1]
    beq = _BE // 4
    nb = n_edges // _BE
    pairs4 = (pair_indices.astype(jnp.int32)
              .reshape(2, nb, beq, 4).transpose(0, 1, 3, 2).reshape(-1))
    d4 = d_ij.reshape(nb, beq, 4).transpose(0, 2, 1)
    zi, zj = _sc_zgather(pairs4, atomic_numbers.astype(jnp.int32))
    embp = jnp.pad(emb_table, ((0, 96 - emb_table.shape[0]), (0, 0)))
    out = pl.pallas_call(
        _tc_body,
        grid=(nb,),
        in_specs=[
            pl.BlockSpec((1, 4, beq), lambda i: (i, 0, 0)),
            pl.BlockSpec((1, 4, beq), lambda i: (i, 0, 0)),
            pl.BlockSpec((1, 4, beq), lambda i: (i, 0, 0)),
            pl.BlockSpec((16, 1), lambda i: (0, 0)),
            pl.BlockSpec((96, emb_size), lambda i: (0, 0)),
            pl.BlockSpec((16, emb_size), lambda i: (0, 0)),
            pl.BlockSpec((emb_size, 1), lambda i: (0, 0)),
            pl.BlockSpec((96, emb_size), lambda i: (0, 0)),
            pl.BlockSpec((emb_size, 1), lambda i: (0, 0)),
        ],
        out_specs=pl.BlockSpec((beq, 128), lambda i: (i, 0)),
        out_shape=jax.ShapeDtypeStruct((n_edges // 4, 128), jnp.float32),
    )(d4, zi.reshape(nb, 4, beq), zj.reshape(nb, 4, beq),
      frequencies.reshape(16, 1), embp.astype(jnp.bfloat16),
      W_rbf.astype(jnp.bfloat16), b_rbf.reshape(emb_size, 1),
      W_dense.astype(jnp.bfloat16), b_dense.reshape(emb_size, 1))
    return out.reshape(n_edges, emb_size)


# speedup vs baseline: 22.9952x; 1.5262x over previous
import jax
import jax.numpy as jnp
from jax import lax
from jax.experimental import pallas as pl
from jax.experimental.pallas import tpu as pltpu
from jax.experimental.pallas import tpu_sc as plsc

_RADIAL_CUTOFF = 5.0
_P = 6.0
_EA = -((_P + 1.0) * (_P + 2.0)) / 2.0
_EB = _P * (_P + 2.0)
_EC = -_P * (_P + 1.0) / 2.0

_BE = 12800
_SCCH = 3200


def _sc_stage(pairs, d_flat, atoms):
    n_edges = d_flat.shape[0]
    n_atoms = atoms.shape[0]
    info = plsc.get_sparse_core_info()
    nw = info.num_cores * info.num_subcores
    nb = n_edges // _BE
    bpw = (nb + nw - 1) // nw
    cpb = _BE // _SCCH
    run = _SCCH // 4
    beq = _BE // 4

    def body(pairs_hbm, d_hbm, atoms_hbm, zi_hbm, zj_hbm, d4_hbm,
             atoms_v, ii_v, ij_v, di_v, zi_s, zj_s, d_s):
        wid = lax.axis_index("s") * info.num_cores + lax.axis_index("c")
        pltpu.sync_copy(atoms_hbm, atoms_v)
        lane4 = lax.iota(jnp.int32, 16) * 4

        def chunk_body(u, _):
            b = wid + nw * (u // cpb)
            c = u % cpb

            @pl.when(b < nb)
            def _():
                ebase = b * _BE + c * _SCCH
                pltpu.sync_copy(pairs_hbm.at[pl.ds(ebase, _SCCH)], ii_v)
                pltpu.sync_copy(pairs_hbm.at[pl.ds(n_edges + ebase, _SCCH)],
                                ij_v)
                pltpu.sync_copy(d_hbm.at[pl.ds(ebase, _SCCH)], di_v)
                for a in range(4):
                    for g in range(run // 16):
                        pos = lane4 + (g * 64 + a)
                        iv = plsc.load_gather(ii_v, [pos])
                        jv = plsc.load_gather(ij_v, [pos])
                        sl = pl.ds(a * run + g * 16, 16)
                        zi_s[sl] = plsc.load_gather(atoms_v, [iv])
                        zj_s[sl] = plsc.load_gather(atoms_v, [jv])
                        d_s[sl] = plsc.load_gather(di_v, [pos])
                for a in range(4):
                    src = pl.ds(a * run, run)
                    dst = pl.ds(b * _BE + a * beq + c * run, run)
                    pltpu.sync_copy(zi_s.at[src], zi_hbm.at[dst])
                    pltpu.sync_copy(zj_s.at[src], zj_hbm.at[dst])
                    pltpu.sync_copy(d_s.at[src], d4_hbm.at[dst])
            return 0

        lax.fori_loop(0, bpw * cpb, chunk_body, 0)

    mesh = plsc.VectorSubcoreMesh(core_axis_name="c", subcore_axis_name="s")
    fn = pl.kernel(
        body,
        mesh=mesh,
        compiler_params=pltpu.CompilerParams(needs_layout_passes=False),
        out_type=[
            jax.ShapeDtypeStruct((n_edges,), jnp.int32),
            jax.ShapeDtypeStruct((n_edges,), jnp.int32),
            jax.ShapeDtypeStruct((n_edges,), jnp.float32),
        ],
        scratch_types=[
            pltpu.VMEM((n_atoms,), jnp.int32),
            pltpu.VMEM((_SCCH,), jnp.int32),
            pltpu.VMEM((_SCCH,), jnp.int32),
            pltpu.VMEM((_SCCH,), jnp.float32),
            pltpu.VMEM((_SCCH,), jnp.int32),
            pltpu.VMEM((_SCCH,), jnp.int32),
            pltpu.VMEM((_SCCH,), jnp.float32),
        ],
    )
    return fn(pairs, d_flat, atoms)


_S1 = 9.9993896014e-01
_S3 = -1.6614390484e-01
_S5 = 7.6898124879e-03
_PI = 3.14159265358979
_INV_PI = 1.0 / _PI
_CT = (((0,), (0,)), ((), ()))


def _dot(a, b):
    return lax.dot_general(a, b, _CT, preferred_element_type=jnp.float32)


def _silu(v):
    return (0.5 * v) * (1.0 + jnp.tanh(0.5 * v))


def _tc_body(d_ref, zi_ref, zj_ref, freq_ref, emb_ref, wr_ref, br_ref,
             wd_ref, bd_ref, out_ref):
    f32 = jnp.float32
    beq = d_ref.shape[2]
    emb = emb_ref[:, :]
    t1 = jnp.dot(emb, wd_ref[0:32, :],
                 preferred_element_type=f32).astype(jnp.bfloat16)
    t2 = jnp.dot(emb, wd_ref[32:64, :],
                 preferred_element_type=f32).astype(jnp.bfloat16)
    iota = lax.broadcasted_iota(jnp.int32, (96, beq), 0)
    parts = []
    for a in range(4):
        d = d_ref[0, a:a + 1, :]
        x = d * (1.0 / _RADIAL_CUTOFF)
        x2 = x * x
        x5 = x2 * x2 * x
        env = 1.0 / x + x5 * (_EA + x * (_EB + x * _EC))
        env = jnp.where(x < 1.0, env, jnp.zeros_like(env))
        tT = jnp.broadcast_to(x, (16, beq)) * freq_ref[:, :]
        n = (tT * _INV_PI + 0.5).astype(jnp.int32)
        r = tT - n.astype(f32) * _PI
        u = r * r
        s = r * (_S1 + u * (_S3 + u * _S5))
        s = jnp.where((n & 1) == 0, s, -s)
        rbfT = (jnp.broadcast_to(env, s.shape) * s).astype(jnp.bfloat16)
        hT = _silu(_dot(wr_ref[:, :], rbfT) + br_ref[:, :])
        oiT = (zi_ref[0, a:a + 1, :] == iota).astype(jnp.bfloat16)
        ojT = (zj_ref[0, a:a + 1, :] == iota).astype(jnp.bfloat16)
        accT = (_dot(t1, oiT) + _dot(t2, ojT)
                + _dot(wd_ref[64:96, :], hT.astype(jnp.bfloat16))
                + bd_ref[:, :])
        parts.append(_silu(accT))
    packedT = jnp.concatenate(parts, axis=0)
    out_ref[:, :] = packedT.T


def kernel(atomic_numbers, d_ij, pair_indices, frequencies, emb_table,
           W_rbf, b_rbf, W_dense, b_dense):
    n_edges = d_ij.shape[0]
    emb_size = emb_table.shape[1]
    beq = _BE // 4
    nb = n_edges // _BE
    zi, zj, d4 = _sc_stage(pair_indices.astype(jnp.int32).reshape(-1),
                           d_ij.reshape(-1), atomic_numbers.astype(jnp.int32))
    d4 = d4.reshape(nb, 4, beq)
    embp = jnp.pad(emb_table, ((0, 96 - emb_table.shape[0]), (0, 0)))
    out = pl.pallas_call(
        _tc_body,
        grid=(nb,),
        in_specs=[
            pl.BlockSpec((1, 4, beq), lambda i: (i, 0, 0)),
            pl.BlockSpec((1, 4, beq), lambda i: (i, 0, 0)),
            pl.BlockSpec((1, 4, beq), lambda i: (i, 0, 0)),
            pl.BlockSpec((16, 1), lambda i: (0, 0)),
            pl.BlockSpec((96, emb_size), lambda i: (0, 0)),
            pl.BlockSpec((16, emb_size), lambda i: (0, 0)),
            pl.BlockSpec((emb_size, 1), lambda i: (0, 0)),
            pl.BlockSpec((96, emb_size), lambda i: (0, 0)),
            pl.BlockSpec((emb_size, 1), lambda i: (0, 0)),
        ],
        out_specs=pl.BlockSpec((beq, 128), lambda i: (i, 0)),
        out_shape=jax.ShapeDtypeStruct((n_edges // 4, 128), jnp.float32),
    )(d4, zi.reshape(nb, 4, beq), zj.reshape(nb, 4, beq),
      frequencies.reshape(16, 1), embp.astype(jnp.bfloat16),
      W_rbf.astype(jnp.bfloat16), b_rbf.reshape(emb_size, 1),
      W_dense.astype(jnp.bfloat16), b_dense.reshape(emb_size, 1))
    return out.reshape(n_edges, emb_size)


# speedup vs baseline: 23.6401x; 1.0280x over previous
import jax
import jax.numpy as jnp
from jax import lax
from jax.experimental import pallas as pl
from jax.experimental.pallas import tpu as pltpu
from jax.experimental.pallas import tpu_sc as plsc

_RADIAL_CUTOFF = 5.0
_P = 6.0
_EA = -((_P + 1.0) * (_P + 2.0)) / 2.0
_EB = _P * (_P + 2.0)
_EC = -_P * (_P + 1.0) / 2.0

_BE = 12800
_SCCH = 3200


def _sc_stage(pairs, d_flat, atoms):
    n_edges = pairs.shape[1]
    n_atoms = atoms.shape[0]
    info = plsc.get_sparse_core_info()
    nw = info.num_cores * info.num_subcores
    nb = n_edges // _BE
    bpw = (nb + nw - 1) // nw
    cpb = _BE // _SCCH
    run = _SCCH // 4
    beq = _BE // 4

    def body(pairs_hbm, d_hbm, atoms_hbm, zi_hbm, zj_hbm, d4_hbm,
             atoms_v, ij2_v, di_v, zi_s, zj_s, d_s):
        wid = lax.axis_index("s") * info.num_cores + lax.axis_index("c")
        pltpu.sync_copy(atoms_hbm, atoms_v)
        lane4 = lax.iota(jnp.int32, 16) * 4
        row0 = jnp.zeros((16,), jnp.int32)
        row1 = row0 + 1

        def chunk_body(u, _):
            b = wid + nw * (u // cpb)
            c = u % cpb

            @pl.when(b < nb)
            def _():
                ebase = b * _BE + c * _SCCH
                pltpu.sync_copy(pairs_hbm.at[:, pl.ds(ebase, _SCCH)], ij2_v)
                pltpu.sync_copy(d_hbm.at[pl.ds(ebase, _SCCH)], di_v)
                for a in range(4):
                    for g in range(run // 16):
                        pos = lane4 + (g * 64 + a)
                        iv = plsc.load_gather(ij2_v, [row0, pos])
                        jv = plsc.load_gather(ij2_v, [row1, pos])
                        sl = pl.ds(a * run + g * 16, 16)
                        zi_s[sl] = plsc.load_gather(atoms_v, [iv])
                        zj_s[sl] = plsc.load_gather(atoms_v, [jv])
                        d_s[sl] = plsc.load_gather(di_v, [pos])
                for a in range(4):
                    src = pl.ds(a * run, run)
                    dst = pl.ds(b * _BE + a * beq + c * run, run)
                    pltpu.sync_copy(zi_s.at[src], zi_hbm.at[dst])
                    pltpu.sync_copy(zj_s.at[src], zj_hbm.at[dst])
                    pltpu.sync_copy(d_s.at[src], d4_hbm.at[dst])
            return 0

        lax.fori_loop(0, bpw * cpb, chunk_body, 0)

    mesh = plsc.VectorSubcoreMesh(core_axis_name="c", subcore_axis_name="s")
    fn = pl.kernel(
        body,
        mesh=mesh,
        compiler_params=pltpu.CompilerParams(needs_layout_passes=False),
        out_type=[
            jax.ShapeDtypeStruct((n_edges,), jnp.int32),
            jax.ShapeDtypeStruct((n_edges,), jnp.int32),
            jax.ShapeDtypeStruct((n_edges,), jnp.float32),
        ],
        scratch_types=[
            pltpu.VMEM((n_atoms,), jnp.int32),
            pltpu.VMEM((2, _SCCH), jnp.int32),
            pltpu.VMEM((_SCCH,), jnp.float32),
            pltpu.VMEM((_SCCH,), jnp.int32),
            pltpu.VMEM((_SCCH,), jnp.int32),
            pltpu.VMEM((_SCCH,), jnp.float32),
        ],
    )
    return fn(pairs, d_flat, atoms)


_S1 = 9.9993896014e-01
_S3 = -1.6614390484e-01
_S5 = 7.6898124879e-03
_PI = 3.14159265358979
_INV_PI = 1.0 / _PI
_CT = (((0,), (0,)), ((), ()))


def _dot(a, b):
    return lax.dot_general(a, b, _CT, preferred_element_type=jnp.float32)


def _silu(v):
    return (0.5 * v) * (1.0 + jnp.tanh(0.5 * v))


def _tc_body(d_ref, zi_ref, zj_ref, freq_ref, emb_ref, wr_ref, br_ref,
             wd_ref, bd_ref, out_ref):
    f32 = jnp.float32
    beq = d_ref.shape[2]
    emb = emb_ref[:, :]
    t1 = jnp.dot(emb, wd_ref[0:32, :],
                 preferred_element_type=f32).astype(jnp.bfloat16)
    t2 = jnp.dot(emb, wd_ref[32:64, :],
                 preferred_element_type=f32).astype(jnp.bfloat16)
    iota = lax.broadcasted_iota(jnp.int32, (96, beq), 0)
    parts = []
    for a in range(4):
        d = d_ref[0, a:a + 1, :]
        x = d * (1.0 / _RADIAL_CUTOFF)
        x2 = x * x
        x5 = x2 * x2 * x
        env = 1.0 / x + x5 * (_EA + x * (_EB + x * _EC))
        env = jnp.where(x < 1.0, env, jnp.zeros_like(env))
        tT = jnp.broadcast_to(x, (16, beq)) * freq_ref[:, :]
        n = (tT * _INV_PI + 0.5).astype(jnp.int32)
        r = tT - n.astype(f32) * _PI
        u = r * r
        s = r * (_S1 + u * (_S3 + u * _S5))
        s = jnp.where((n & 1) == 0, s, -s)
        rbfT = (jnp.broadcast_to(env, s.shape) * s).astype(jnp.bfloat16)
        hT = _silu(_dot(wr_ref[:, :], rbfT) + br_ref[:, :])
        oiT = (zi_ref[0, a:a + 1, :] == iota).astype(jnp.bfloat16)
        ojT = (zj_ref[0, a:a + 1, :] == iota).astype(jnp.bfloat16)
        accT = (_dot(t1, oiT) + _dot(t2, ojT)
                + _dot(wd_ref[64:96, :], hT.astype(jnp.bfloat16))
                + bd_ref[:, :])
        parts.append(_silu(accT))
    packedT = jnp.concatenate(parts, axis=0)
    out_ref[:, :] = packedT.T


def kernel(atomic_numbers, d_ij, pair_indices, frequencies, emb_table,
           W_rbf, b_rbf, W_dense, b_dense):
    n_edges = d_ij.shape[0]
    emb_size = emb_table.shape[1]
    beq = _BE // 4
    nb = n_edges // _BE
    zi, zj, d4 = _sc_stage(pair_indices.astype(jnp.int32),
                           d_ij.reshape(-1), atomic_numbers.astype(jnp.int32))
    d4 = d4.reshape(nb, 4, beq)
    embp = jnp.pad(emb_table, ((0, 96 - emb_table.shape[0]), (0, 0)))
    out = pl.pallas_call(
        _tc_body,
        grid=(nb,),
        in_specs=[
            pl.BlockSpec((1, 4, beq), lambda i: (i, 0, 0)),
            pl.BlockSpec((1, 4, beq), lambda i: (i, 0, 0)),
            pl.BlockSpec((1, 4, beq), lambda i: (i, 0, 0)),
            pl.BlockSpec((16, 1), lambda i: (0, 0)),
            pl.BlockSpec((96, emb_size), lambda i: (0, 0)),
            pl.BlockSpec((16, emb_size), lambda i: (0, 0)),
            pl.BlockSpec((emb_size, 1), lambda i: (0, 0)),
            pl.BlockSpec((96, emb_size), lambda i: (0, 0)),
            pl.BlockSpec((emb_size, 1), lambda i: (0, 0)),
        ],
        out_specs=pl.BlockSpec((beq, 128), lambda i: (i, 0)),
        out_shape=jax.ShapeDtypeStruct((n_edges // 4, 128), jnp.float32),
    )(d4, zi.reshape(nb, 4, beq), zj.reshape(nb, 4, beq),
      frequencies.reshape(16, 1), embp.astype(jnp.bfloat16),
      W_rbf.astype(jnp.bfloat16), b_rbf.reshape(emb_size, 1),
      W_dense.astype(jnp.bfloat16), b_dense.reshape(emb_size, 1))
    return out.reshape(n_edges, emb_size)


# speedup vs baseline: 30.0514x; 1.2712x over previous
import jax
import jax.numpy as jnp
from jax import lax
from jax.experimental import pallas as pl
from jax.experimental.pallas import tpu as pltpu
from jax.experimental.pallas import tpu_sc as plsc

_RADIAL_CUTOFF = 5.0
_P = 6.0
_EA = -((_P + 1.0) * (_P + 2.0)) / 2.0
_EB = _P * (_P + 2.0)
_EC = -_P * (_P + 1.0) / 2.0

_BE = 12800
_SCCH = 3200


def _sc_stage(pairs, d_flat, atoms):
    n_edges = pairs.shape[1]
    n_atoms = atoms.shape[0]
    info = plsc.get_sparse_core_info()
    nw = info.num_cores * info.num_subcores
    nb = n_edges // _BE
    bpw = (nb + nw - 1) // nw
    cpb = _BE // _SCCH
    run = _SCCH // 4
    beq = _BE // 4

    def body(pairs_hbm, d_hbm, atoms_hbm, zi_hbm, zj_hbm, d4_hbm,
             atoms_v, ij2_v, di_v, zi_s, zj_s, d_s):
        wid = lax.axis_index("s") * info.num_cores + lax.axis_index("c")
        pltpu.sync_copy(atoms_hbm, atoms_v)
        lane16 = lax.iota(jnp.int32, 16)
        row0 = jnp.zeros((16,), jnp.int32)
        row1 = row0 + 1

        def chunk_body(u, _):
            b = wid + nw * (u // cpb)
            c = u % cpb

            @pl.when(b < nb)
            def _():
                ebase = b * _BE + c * _SCCH
                pltpu.sync_copy(pairs_hbm.at[:, pl.ds(ebase, _SCCH)], ij2_v)
                pltpu.sync_copy(d_hbm.at[pl.ds(ebase, _SCCH)], di_v)
                for g in range(_SCCH // 16):
                    pos = lane16 + g * 16
                    iv = plsc.load_gather(ij2_v, [row0, pos])
                    jv = plsc.load_gather(ij2_v, [row1, pos])
                    sl = pl.ds(g * 16, 16)
                    zi_s[sl] = plsc.load_gather(atoms_v, [iv])
                    zj_s[sl] = plsc.load_gather(atoms_v, [jv])
                dst = pl.ds(ebase, _SCCH)
                pltpu.sync_copy(zi_s, zi_hbm.at[dst])
                pltpu.sync_copy(zj_s, zj_hbm.at[dst])
                pltpu.sync_copy(di_v, d4_hbm.at[dst])
            return 0

        lax.fori_loop(0, bpw * cpb, chunk_body, 0)

    mesh = plsc.VectorSubcoreMesh(core_axis_name="c", subcore_axis_name="s")
    fn = pl.kernel(
        body,
        mesh=mesh,
        compiler_params=pltpu.CompilerParams(needs_layout_passes=False),
        out_type=[
            jax.ShapeDtypeStruct((n_edges,), jnp.int32),
            jax.ShapeDtypeStruct((n_edges,), jnp.int32),
            jax.ShapeDtypeStruct((n_edges,), jnp.float32),
        ],
        scratch_types=[
            pltpu.VMEM((n_atoms,), jnp.int32),
            pltpu.VMEM((2, _SCCH), jnp.int32),
            pltpu.VMEM((_SCCH,), jnp.float32),
            pltpu.VMEM((_SCCH,), jnp.int32),
            pltpu.VMEM((_SCCH,), jnp.int32),
            pltpu.VMEM((_SCCH,), jnp.float32),
        ],
    )
    return fn(pairs, d_flat, atoms)


_S1 = 9.9993896014e-01
_S3 = -1.6614390484e-01
_S5 = 7.6898124879e-03
_PI = 3.14159265358979
_INV_PI = 1.0 / _PI
_CT = (((0,), (0,)), ((), ()))


def _dot(a, b):
    return lax.dot_general(a, b, _CT, preferred_element_type=jnp.float32)


def _silu(v):
    return (0.5 * v) * (1.0 + jnp.tanh(0.5 * v))


def _tc_body(d_ref, zi_ref, zj_ref, freq_ref, emb_ref, wr_ref, br_ref,
             wd_ref, bd_ref, out_ref):
    f32 = jnp.float32
    beq = d_ref.shape[2]
    emb = emb_ref[:, :]
    t1 = jnp.dot(emb, wd_ref[0:32, :],
                 preferred_element_type=f32).astype(jnp.bfloat16)
    t2 = jnp.dot(emb, wd_ref[32:64, :],
                 preferred_element_type=f32).astype(jnp.bfloat16)
    iota = lax.broadcasted_iota(jnp.int32, (96, beq), 0)
    parts = []
    for a in range(4):
        d = d_ref[0, a:a + 1, :]
        x = d * (1.0 / _RADIAL_CUTOFF)
        x2 = x * x
        x5 = x2 * x2 * x
        env = 1.0 / x + x5 * (_EA + x * (_EB + x * _EC))
        env = jnp.where(x < 1.0, env, jnp.zeros_like(env))
        tT = jnp.broadcast_to(x, (16, beq)) * freq_ref[:, :]
        n = (tT * _INV_PI + 0.5).astype(jnp.int32)
        r = tT - n.astype(f32) * _PI
        u = r * r
        s = r * (_S1 + u * (_S3 + u * _S5))
        s = jnp.where((n & 1) == 0, s, -s)
        rbfT = (jnp.broadcast_to(env, s.shape) * s).astype(jnp.bfloat16)
        hT = _silu(_dot(wr_ref[:, :], rbfT) + br_ref[:, :])
        oiT = (zi_ref[0, a:a + 1, :] == iota).astype(jnp.bfloat16)
        ojT = (zj_ref[0, a:a + 1, :] == iota).astype(jnp.bfloat16)
        accT = (_dot(t1, oiT) + _dot(t2, ojT)
                + _dot(wd_ref[64:96, :], hT.astype(jnp.bfloat16))
                + bd_ref[:, :])
        parts.append(_silu(accT))
    for a in range(4):
        out_ref[pl.ds(a * beq, beq), :] = parts[a].T


def kernel(atomic_numbers, d_ij, pair_indices, frequencies, emb_table,
           W_rbf, b_rbf, W_dense, b_dense):
    n_edges = d_ij.shape[0]
    emb_size = emb_table.shape[1]
    beq = _BE // 4
    nb = n_edges // _BE
    zi, zj, d4 = _sc_stage(pair_indices.astype(jnp.int32),
                           d_ij.reshape(-1), atomic_numbers.astype(jnp.int32))
    d4 = d4.reshape(nb, 4, beq)
    embp = jnp.pad(emb_table, ((0, 96 - emb_table.shape[0]), (0, 0)))
    out = pl.pallas_call(
        _tc_body,
        grid=(nb,),
        in_specs=[
            pl.BlockSpec((1, 4, beq), lambda i: (i, 0, 0)),
            pl.BlockSpec((1, 4, beq), lambda i: (i, 0, 0)),
            pl.BlockSpec((1, 4, beq), lambda i: (i, 0, 0)),
            pl.BlockSpec((16, 1), lambda i: (0, 0)),
            pl.BlockSpec((96, emb_size), lambda i: (0, 0)),
            pl.BlockSpec((16, emb_size), lambda i: (0, 0)),
            pl.BlockSpec((emb_size, 1), lambda i: (0, 0)),
            pl.BlockSpec((96, emb_size), lambda i: (0, 0)),
            pl.BlockSpec((emb_size, 1), lambda i: (0, 0)),
        ],
        out_specs=pl.BlockSpec((_BE, emb_size), lambda i: (i, 0)),
        out_shape=jax.ShapeDtypeStruct((n_edges, emb_size), jnp.float32),
    )(d4, zi.reshape(nb, 4, beq), zj.reshape(nb, 4, beq),
      frequencies.reshape(16, 1), embp.astype(jnp.bfloat16),
      W_rbf.astype(jnp.bfloat16), b_rbf.reshape(emb_size, 1),
      W_dense.astype(jnp.bfloat16), b_dense.reshape(emb_size, 1))
    return out
